# baseline (speedup 1.0000x reference)
import functools

import jax
import jax.numpy as jnp
from jax import lax
from jax.experimental import pallas as pl
from jax.experimental.pallas import tpu as pltpu
from jax.experimental.pallas import tpu_sc as plsc

B, S = 4, 2048
H = 768
Q = H // 4
VOCAB = 100000
NUMV = 12
MAXCELL = 256
NFMT = 11
EPS = 1e-12

N = B * S
R = 4 * N
NW = 32
ROWS_PER_W = R // NW
CHUNK = 128
NJ = ROWS_PER_W // CHUNK
QV = Q // 16


def _sc_gather_sum(tok_idx, ord_idx, num_idx, tok_w192, ord_w192, num_w):
    mesh = plsc.VectorSubcoreMesh(core_axis_name="c", subcore_axis_name="s")
    info = plsc.get_sparse_core_info()
    nc = info.num_cores

    @functools.partial(
        pl.kernel,
        out_type=jax.ShapeDtypeStruct((R, Q), jnp.float32),
        mesh=mesh,
        scratch_types=[
            pltpu.VMEM((NJ, CHUNK), jnp.int32),
            pltpu.VMEM((NJ, CHUNK), jnp.int32),
            pltpu.VMEM((NJ, CHUNK), jnp.int32),
            pltpu.VMEM((CHUNK, Q), jnp.float32),
            pltpu.VMEM((CHUNK, Q), jnp.float32),
            pltpu.VMEM((CHUNK, Q), jnp.float32),
            pltpu.SemaphoreType.DMA,
            pltpu.SemaphoreType.DMA,
            pltpu.SemaphoreType.DMA,
        ],
        compiler_params=pltpu.CompilerParams(use_tc_tiling_on_sc=False),
    )
    def k(ti_h, oi_h, ni_h, tw_h, ow_h, nw_h, out_h,
          ti_v, oi_v, ni_v, bt, bo, bn, s1, s2, s3):
        wid = lax.axis_index("s") * nc + lax.axis_index("c")
        base = wid * ROWS_PER_W
        pltpu.sync_copy(ti_h.at[wid], ti_v)
        pltpu.sync_copy(oi_h.at[wid], oi_v)
        pltpu.sync_copy(ni_h.at[wid], ni_v)
        for j in range(NJ):
            d1 = pltpu.async_copy(tw_h.at[ti_v.at[j]], bt, s1)
            d2 = pltpu.async_copy(ow_h.at[oi_v.at[j]], bo, s2)
            d3 = pltpu.async_copy(nw_h.at[ni_v.at[j]], bn, s3)
            d1.wait()
            d2.wait()
            d3.wait()

            def add_row(r, _):
                for i in range(QV):
                    sl = pl.ds(i * 16, 16)
                    bt[r, sl] = bt[r, sl] + bo[r, sl] + bn[r, sl]
                return 0

            lax.fori_loop(0, CHUNK, add_row, 0)
            pltpu.sync_copy(bt, out_h.at[pl.ds(base + j * CHUNK, CHUNK)])

    return k(tok_idx, ord_idx, num_idx, tok_w192, ord_w192, num_w)


def _tc_finish(summed, fvec, fwt, gamma, beta):
    BLK = 512
    grid = (N // BLK,)

    def body(s_ref, f_ref, w_ref, g_ref, b_ref, o_ref):
        e = s_ref[...] + jnp.dot(f_ref[...], w_ref[...],
                                 preferred_element_type=jnp.float32)
        mean = jnp.mean(e, axis=-1, keepdims=True)
        c = e - mean
        var = jnp.mean(c * c, axis=-1, keepdims=True)
        o_ref[...] = c * lax.rsqrt(var + EPS) * g_ref[...] + b_ref[...]

    return pl.pallas_call(
        body,
        grid=grid,
        in_specs=[
            pl.BlockSpec((BLK, H), lambda i: (i, 0)),
            pl.BlockSpec((BLK, NFMT), lambda i: (i, 0)),
            pl.BlockSpec((NFMT, H), lambda i: (0, 0)),
            pl.BlockSpec((1, H), lambda i: (0, 0)),
            pl.BlockSpec((1, H), lambda i: (0, 0)),
        ],
        out_specs=pl.BlockSpec((BLK, H), lambda i: (i, 0)),
        out_shape=jax.ShapeDtypeStruct((N, H), jnp.float32),
    )(summed, fvec, fwt, gamma, beta)


def kernel(token_id, num_mag, num_pre, num_top, num_low, order, format_vec,
           token_W, mag_W, pre_W, top_W, low_W, order_W, format_W,
           ln_gamma, ln_beta):
    q4 = jnp.arange(4, dtype=jnp.int32)
    tok_idx = (4 * token_id.reshape(-1, 1).astype(jnp.int32) + q4)
    ord_idx = (4 * order.reshape(-1, 1).astype(jnp.int32) + q4)
    num_idx = jnp.stack(
        [num_mag.reshape(-1).astype(jnp.int32),
         num_pre.reshape(-1).astype(jnp.int32) + NUMV,
         num_top.reshape(-1).astype(jnp.int32) + 2 * NUMV,
         num_low.reshape(-1).astype(jnp.int32) + 3 * NUMV], axis=-1)
    tok_idx = tok_idx.reshape(NW, NJ, CHUNK)
    ord_idx = ord_idx.reshape(NW, NJ, CHUNK)
    num_idx = num_idx.reshape(NW, NJ, CHUNK)

    tok_w192 = token_W.reshape(4 * VOCAB, Q)
    ord_w192 = order_W.reshape(4 * MAXCELL, Q)
    num_w = jnp.concatenate([mag_W, pre_W, top_W, low_W], axis=0)

    summed = _sc_gather_sum(tok_idx, ord_idx, num_idx,
                            tok_w192, ord_w192, num_w)
    summed = summed.reshape(N, H)

    out = _tc_finish(summed, format_vec.reshape(N, NFMT), format_W.T,
                     ln_gamma.reshape(1, H), ln_beta.reshape(1, H))
    return out.reshape(B, S, H)


# speedup vs baseline: 1.0261x; 1.0261x over previous
import functools

import jax
import jax.numpy as jnp
from jax import lax
from jax.experimental import pallas as pl
from jax.experimental.pallas import tpu as pltpu
from jax.experimental.pallas import tpu_sc as plsc

B, S = 4, 2048
H = 768
Q = H // 4
VOCAB = 100000
NUMV = 12
MAXCELL = 256
NFMT = 11
EPS = 1e-12

N = B * S
R = 4 * N
NW = 32
ROWS_PER_W = R // NW
CHUNK = 128
NJ = ROWS_PER_W // CHUNK
QV = Q // 16


def _sc_gather_sum(tok_idx, ord_idx, num_idx, tok_w192, ord_w192, num_w):
    mesh = plsc.VectorSubcoreMesh(core_axis_name="c", subcore_axis_name="s")
    info = plsc.get_sparse_core_info()
    nc = info.num_cores

    @functools.partial(
        pl.kernel,
        out_type=jax.ShapeDtypeStruct((R, Q), jnp.float32),
        mesh=mesh,
        scratch_types=[
            pltpu.VMEM((NJ, CHUNK), jnp.int32),
            pltpu.VMEM((NJ, CHUNK), jnp.int32),
            pltpu.VMEM((NJ, CHUNK), jnp.int32),
            [pltpu.VMEM((CHUNK, Q), jnp.float32)] * 2,
            [pltpu.VMEM((CHUNK, Q), jnp.float32)] * 2,
            pltpu.VMEM((4 * NUMV, Q), jnp.float32),
            [pltpu.SemaphoreType.DMA] * 2,
            [pltpu.SemaphoreType.DMA] * 2,
        ],
        compiler_params=pltpu.CompilerParams(use_tc_tiling_on_sc=False),
    )
    def k(ti_h, oi_h, ni_h, tw_h, ow_h, nw_h, out_h,
          ti_v, oi_v, ni_vm, bt, bo, ntab, gsem, osem):
        wid = lax.axis_index("s") * nc + lax.axis_index("c")
        base = wid * ROWS_PER_W
        pltpu.sync_copy(ti_h.at[wid], ti_v)
        pltpu.sync_copy(oi_h.at[wid], oi_v)
        pltpu.sync_copy(ni_h.at[wid], ni_vm)
        pltpu.sync_copy(nw_h, ntab)

        gd = [None, None]
        od = [None, None]

        def process(j):
            p = j & 1
            for d in gd[p]:
                d.wait()

            def add_rows(rb, _):
                vn = ni_vm[j, pl.ds(rb * 16, 16)]
                for l in range(16):
                    r = rb * 16 + l
                    nidx = vn[l]
                    for i in range(QV):
                        sl = pl.ds(i * 16, 16)
                        bt[p][r, sl] = (bt[p][r, sl] + bo[p][r, sl]
                                        + ntab[nidx, sl])
                return 0

            lax.fori_loop(0, CHUNK // 16, add_rows, 0)
            od[p] = pltpu.async_copy(
                bt[p], out_h.at[pl.ds(base + j * CHUNK, CHUNK)], osem[p])

        for j in range(NJ):
            p = j & 1
            if od[p] is not None:
                od[p].wait()
            gd[p] = [
                pltpu.async_copy(tw_h.at[ti_v.at[j]], bt[p], gsem[p]),
                pltpu.async_copy(ow_h.at[oi_v.at[j]], bo[p], gsem[p]),
            ]
            if j >= 1:
                process(j - 1)
        process(NJ - 1)
        od[(NJ - 1) & 1].wait()
        od[NJ & 1].wait()

    return k(tok_idx, ord_idx, num_idx, tok_w192, ord_w192, num_w)


def _tc_finish(summed, fvec, fwt, gamma, beta):
    BLK = 512
    grid = (N // BLK,)

    def body(s_ref, f_ref, w_ref, g_ref, b_ref, o_ref):
        e = s_ref[...] + jnp.dot(f_ref[...], w_ref[...],
                                 preferred_element_type=jnp.float32)
        mean = jnp.mean(e, axis=-1, keepdims=True)
        c = e - mean
        var = jnp.mean(c * c, axis=-1, keepdims=True)
        o_ref[...] = c * lax.rsqrt(var + EPS) * g_ref[...] + b_ref[...]

    return pl.pallas_call(
        body,
        grid=grid,
        in_specs=[
            pl.BlockSpec((BLK, H), lambda i: (i, 0)),
            pl.BlockSpec((BLK, NFMT), lambda i: (i, 0)),
            pl.BlockSpec((NFMT, H), lambda i: (0, 0)),
            pl.BlockSpec((1, H), lambda i: (0, 0)),
            pl.BlockSpec((1, H), lambda i: (0, 0)),
        ],
        out_specs=pl.BlockSpec((BLK, H), lambda i: (i, 0)),
        out_shape=jax.ShapeDtypeStruct((N, H), jnp.float32),
    )(summed, fvec, fwt, gamma, beta)


def kernel(token_id, num_mag, num_pre, num_top, num_low, order, format_vec,
           token_W, mag_W, pre_W, top_W, low_W, order_W, format_W,
           ln_gamma, ln_beta):
    q4 = jnp.arange(4, dtype=jnp.int32)
    tok_idx = (4 * token_id.reshape(-1, 1).astype(jnp.int32) + q4)
    ord_idx = (4 * order.reshape(-1, 1).astype(jnp.int32) + q4)
    num_idx = jnp.stack(
        [num_mag.reshape(-1).astype(jnp.int32),
         num_pre.reshape(-1).astype(jnp.int32) + NUMV,
         num_top.reshape(-1).astype(jnp.int32) + 2 * NUMV,
         num_low.reshape(-1).astype(jnp.int32) + 3 * NUMV], axis=-1)
    tok_idx = tok_idx.reshape(NW, NJ, CHUNK)
    ord_idx = ord_idx.reshape(NW, NJ, CHUNK)
    num_idx = num_idx.reshape(NW, NJ, CHUNK)

    tok_w192 = token_W.reshape(4 * VOCAB, Q)
    ord_w192 = order_W.reshape(4 * MAXCELL, Q)
    num_w = jnp.concatenate([mag_W, pre_W, top_W, low_W], axis=0)

    summed = _sc_gather_sum(tok_idx, ord_idx, num_idx,
                            tok_w192, ord_w192, num_w)
    summed = summed.reshape(N, H)

    out = _tc_finish(summed, format_vec.reshape(N, NFMT), format_W.T,
                     ln_gamma.reshape(1, H), ln_beta.reshape(1, H))
    return out.reshape(B, S, H)
